# baseline (speedup 1.0000x reference)
import jax
import jax.numpy as jnp
from jax import lax
from jax.experimental import pallas as pl
from jax.experimental.pallas import tpu as pltpu
from jax.experimental.pallas import tpu_sc as plsc

_BM = 1024


_FP = 40
_WROWS = 8
_NBUF = 4


def _sc_gather(table, wp, B, E):
    mesh = plsc.VectorSubcoreMesh(core_axis_name="core", subcore_axis_name="subcore")
    n_steps = B // _WROWS
    blk = _WROWS * _FP
    info = plsc.get_sparse_core_info()
    nw = info.num_cores * info.num_subcores
    spt = n_steps // nw
    NB = _NBUF

    @pl.kernel(
        out_type=jax.ShapeDtypeStruct((n_steps, blk, E), table.dtype),
        mesh=mesh,
        scratch_types=[
            pltpu.VMEM((NB, _WROWS, _FP), jnp.int32),
            pltpu.VMEM((NB, blk, E), jnp.float32),
            pltpu.SemaphoreType.DMA((NB,)),
            pltpu.SemaphoreType.DMA((NB,)),
            pltpu.SemaphoreType.DMA((NB,)),
        ],
        compiler_params=pltpu.CompilerParams(use_tc_tiling_on_sc=False),
    )
    def k(tab_hbm, w_hbm, o_hbm, idx_v, rows_v, isem, gsem, osem):
        wid = lax.axis_index("subcore") * info.num_cores + lax.axis_index("core")
        base = wid * spt

        def idx_cp(s, b):
            return pltpu.make_async_copy(
                w_hbm.at[pl.ds((base + s) * _WROWS, _WROWS), pl.ds(0, _FP)],
                idx_v.at[b], isem.at[b])

        def gath(b, j):
            return pltpu.make_async_copy(
                tab_hbm.at[idx_v.at[b, j]],
                rows_v.at[b, pl.ds(j * _FP, _FP), :],
                gsem.at[b])

        def out_cp(s, b):
            return pltpu.make_async_copy(rows_v.at[b], o_hbm.at[base + s], osem.at[b])

        for b in range(NB):
            idx_cp(b, b).start()

        @pl.loop(0, spt // NB)
        def _(oi):
            for b in range(NB):
                s = oi * NB + b
                bp = (b - 1) % NB
                idx_cp(s, b).wait()

                @pl.when(oi > 0)
                def _():
                    out_cp(s - NB, b).wait()

                for j in range(_WROWS):
                    gath(b, j).start()

                @pl.when(s > 0)
                def _():
                    for j in range(_WROWS):
                        gath(bp, j).wait()

                    @pl.when(s - 1 + NB < spt)
                    def _():
                        idx_cp(s - 1 + NB, bp).start()

                    out_cp(s - 1, bp).start()

        bl = (spt - 1) % NB
        for j in range(_WROWS):
            gath(bl, j).wait()
        out_cp(spt - 1, bl).start()
        for b in range(NB):
            out_cp(spt - NB + b, (spt - NB + b) % NB).wait()

    return k(table, wp)


def _tc_mlp(x, W1, b1, W2, b2):
    B, K = x.shape
    H = W1.shape[1]
    C = W2.shape[1]

    def body(x_ref, W1_ref, b1_ref, W2_ref, b2_ref, o_ref):
        xb = x_ref[...].astype(jnp.bfloat16)
        h = jnp.dot(xb, W1_ref[...], preferred_element_type=jnp.float32)
        h = jnp.maximum(h + b1_ref[...], 0.0).astype(jnp.bfloat16)
        o_ref[...] = jnp.dot(h, W2_ref[...], preferred_element_type=jnp.float32) + b2_ref[...]

    return pl.pallas_call(
        body,
        grid=(B // _BM,),
        in_specs=[
            pl.BlockSpec((_BM, K), lambda i: (i, 0)),
            pl.BlockSpec((K, H), lambda i: (0, 0)),
            pl.BlockSpec((1, H), lambda i: (0, 0)),
            pl.BlockSpec((H, C), lambda i: (0, 0)),
            pl.BlockSpec((1, C), lambda i: (0, 0)),
        ],
        out_specs=pl.BlockSpec((_BM, C), lambda i: (i, 0)),
        out_shape=jax.ShapeDtypeStruct((B, C), jnp.float32),
    )(x, W1.astype(jnp.bfloat16), b1.reshape(1, -1), W2.astype(jnp.bfloat16),
      b2.reshape(1, -1))


def kernel(w, embeddings, W1, b1, W2, b2):
    B, F = w.shape
    V, E = embeddings.shape
    wp = jnp.pad(w.astype(jnp.int32), ((0, 0), (0, 128 - F)))
    x = _sc_gather(embeddings, wp, B, E)
    x = x.reshape(B, _FP * E)
    W1p = jnp.pad(W1, ((0, (_FP - F) * E), (0, 0)))
    return _tc_mlp(x, W1p, b1, W2, b2)


# speedup vs baseline: 2.6415x; 2.6415x over previous
import jax
import jax.numpy as jnp
from jax import lax
from jax.experimental import pallas as pl
from jax.experimental.pallas import tpu as pltpu
from jax.experimental.pallas import tpu_sc as plsc

_BM = 1024


_SUB = 96
_NSUB = 3
_NBUF = 4


def _sc_gather(table, wlin, n_idx, E):
    mesh = plsc.VectorSubcoreMesh(core_axis_name="core", subcore_axis_name="subcore")
    blk = _SUB * _NSUB
    n_steps = n_idx // blk
    info = plsc.get_sparse_core_info()
    nw = info.num_cores * info.num_subcores
    spt = n_steps // nw
    NB = _NBUF

    @pl.kernel(
        out_type=jax.ShapeDtypeStruct((n_steps, blk, E), table.dtype),
        mesh=mesh,
        scratch_types=[
            pltpu.VMEM((NB, blk), jnp.int32),
            pltpu.VMEM((NB, blk, E), jnp.float32),
            pltpu.SemaphoreType.DMA((NB,)),
            pltpu.SemaphoreType.DMA((NB,)),
            pltpu.SemaphoreType.DMA((NB,)),
        ],
        compiler_params=pltpu.CompilerParams(use_tc_tiling_on_sc=False),
    )
    def k(tab_hbm, w_hbm, o_hbm, idx_v, rows_v, isem, gsem, osem):
        wid = lax.axis_index("subcore") * info.num_cores + lax.axis_index("core")
        base = wid * spt

        def idx_cp(s, b):
            return pltpu.make_async_copy(
                w_hbm.at[pl.ds((base + s) * blk, blk)], idx_v.at[b], isem.at[b])

        def gath(b, j):
            return pltpu.make_async_copy(
                tab_hbm.at[idx_v.at[b, pl.ds(j * _SUB, _SUB)]],
                rows_v.at[b, pl.ds(j * _SUB, _SUB), :],
                gsem.at[b])

        def out_cp(s, b):
            return pltpu.make_async_copy(rows_v.at[b], o_hbm.at[base + s], osem.at[b])

        for b in range(NB):
            idx_cp(b, b).start()

        @pl.loop(0, spt // NB)
        def _(oi):
            for b in range(NB):
                s = oi * NB + b
                bp = (b - 1) % NB
                idx_cp(s, b).wait()

                @pl.when(oi > 0)
                def _():
                    out_cp(s - NB, b).wait()

                for j in range(_NSUB):
                    gath(b, j).start()

                @pl.when(s > 0)
                def _():
                    for j in range(_NSUB):
                        gath(bp, j).wait()

                    @pl.when(s - 1 + NB < spt)
                    def _():
                        idx_cp(s - 1 + NB, bp).start()

                    out_cp(s - 1, bp).start()

        bl = (spt - 1) % NB
        for j in range(_NSUB):
            gath(bl, j).wait()
        out_cp(spt - 1, bl).start()
        for b in range(NB):
            out_cp(spt - NB + b, (spt - NB + b) % NB).wait()

    return k(table, wlin)


def _tc_mlp(x, W1, b1, W2, b2):
    B, K = x.shape
    H = W1.shape[1]
    C = W2.shape[1]

    def body(x_ref, W1_ref, b1_ref, W2_ref, b2_ref, o_ref):
        xb = x_ref[...].astype(jnp.bfloat16)
        h = jnp.dot(xb, W1_ref[...], preferred_element_type=jnp.float32)
        h = jnp.maximum(h + b1_ref[...], 0.0).astype(jnp.bfloat16)
        o_ref[...] = jnp.dot(h, W2_ref[...], preferred_element_type=jnp.float32) + b2_ref[...]

    return pl.pallas_call(
        body,
        grid=(B // _BM,),
        in_specs=[
            pl.BlockSpec((_BM, K), lambda i: (i, 0)),
            pl.BlockSpec((K, H), lambda i: (0, 0)),
            pl.BlockSpec((1, H), lambda i: (0, 0)),
            pl.BlockSpec((H, C), lambda i: (0, 0)),
            pl.BlockSpec((1, C), lambda i: (0, 0)),
        ],
        out_specs=pl.BlockSpec((_BM, C), lambda i: (i, 0)),
        out_shape=jax.ShapeDtypeStruct((B, C), jnp.float32),
    )(x, W1.astype(jnp.bfloat16), b1.reshape(1, -1), W2.astype(jnp.bfloat16),
      b2.reshape(1, -1))


def kernel(w, embeddings, W1, b1, W2, b2):
    B, F = w.shape
    V, E = embeddings.shape
    wlin = w.astype(jnp.int32).reshape(-1)
    x = _sc_gather(embeddings, wlin, B * F, E)
    x = x.reshape(B, F * E)
    return _tc_mlp(x, W1, b1, W2, b2)


# speedup vs baseline: 3.0984x; 1.1730x over previous
import jax
import jax.numpy as jnp
from jax import lax
from jax.experimental import pallas as pl
from jax.experimental.pallas import tpu as pltpu
from jax.experimental.pallas import tpu_sc as plsc

_BM = 1024


_SUB = 96
_NSUB = 3
_NBUF = 4


def _sc_gather(table, wlin, n_idx, E):
    mesh = plsc.VectorSubcoreMesh(core_axis_name="core", subcore_axis_name="subcore")
    blk = _SUB * _NSUB
    n_steps = n_idx // blk
    info = plsc.get_sparse_core_info()
    nw = info.num_cores * info.num_subcores
    spt = n_steps // nw
    NB = _NBUF

    @pl.kernel(
        out_type=jax.ShapeDtypeStruct((n_steps, blk, E), table.dtype),
        mesh=mesh,
        scratch_types=[
            pltpu.VMEM((NB, blk), jnp.int32),
            pltpu.VMEM((NB, blk, E), jnp.float32),
            pltpu.SemaphoreType.DMA((NB,)),
            pltpu.SemaphoreType.DMA((NB,)),
            pltpu.SemaphoreType.DMA((NB,)),
        ],
        compiler_params=pltpu.CompilerParams(use_tc_tiling_on_sc=False),
    )
    def k(tab_hbm, w_hbm, o_hbm, idx_v, rows_v, isem, gsem, osem):
        wid = lax.axis_index("subcore") * info.num_cores + lax.axis_index("core")
        base = wid * spt

        def idx_cp(s, b):
            return pltpu.make_async_copy(
                w_hbm.at[pl.ds((base + s) * blk, blk)], idx_v.at[b], isem.at[b])

        def gath(b, j):
            return pltpu.make_async_copy(
                tab_hbm.at[idx_v.at[b, pl.ds(j * _SUB, _SUB)]],
                rows_v.at[b, pl.ds(j * _SUB, _SUB), :],
                gsem.at[b])

        def out_cp(s, b):
            return pltpu.make_async_copy(rows_v.at[b], o_hbm.at[base + s], osem.at[b])

        for b in range(NB):
            idx_cp(b, b).start()

        @pl.loop(0, spt // NB)
        def _(oi):
            for b in range(NB):
                s = oi * NB + b
                bp = (b - 1) % NB
                idx_cp(s, b).wait()

                @pl.when(oi > 0)
                def _():
                    out_cp(s - NB, b).wait()

                for j in range(_NSUB):
                    gath(b, j).start()

                @pl.when(s > 0)
                def _():
                    for j in range(_NSUB):
                        gath(bp, j).wait()

                    @pl.when(s - 1 + NB < spt)
                    def _():
                        idx_cp(s - 1 + NB, bp).start()

                    out_cp(s - 1, bp).start()

        bl = (spt - 1) % NB
        for j in range(_NSUB):
            gath(bl, j).wait()
        out_cp(spt - 1, bl).start()
        for b in range(NB):
            out_cp(spt - NB + b, (spt - NB + b) % NB).wait()

    return k(table, wlin)


def _tc_mlp(x2d, W1, b1, W2, b2):
    K, H = W1.shape
    C = W2.shape[1]
    G = K // 128
    B = x2d.shape[0] // G

    def body(x_ref, W1_ref, b1_ref, W2_ref, b2_ref, o_ref):
        acc = jnp.zeros((_BM, H), jnp.float32)
        for g in range(G):
            xg = x_ref[pl.Slice(g, _BM, G), :].astype(jnp.bfloat16)
            acc += jnp.dot(xg, W1_ref[pl.ds(g * 128, 128), :],
                           preferred_element_type=jnp.float32)
        h = jnp.maximum(acc + b1_ref[...], 0.0).astype(jnp.bfloat16)
        o_ref[...] = jnp.dot(h, W2_ref[...], preferred_element_type=jnp.float32) + b2_ref[...]

    return pl.pallas_call(
        body,
        grid=(B // _BM,),
        in_specs=[
            pl.BlockSpec((_BM * G, 128), lambda i: (i, 0)),
            pl.BlockSpec((K, H), lambda i: (0, 0)),
            pl.BlockSpec((1, H), lambda i: (0, 0)),
            pl.BlockSpec((H, C), lambda i: (0, 0)),
            pl.BlockSpec((1, C), lambda i: (0, 0)),
        ],
        out_specs=pl.BlockSpec((_BM, C), lambda i: (i, 0)),
        out_shape=jax.ShapeDtypeStruct((B, C), jnp.float32),
    )(x2d, W1.astype(jnp.bfloat16), b1.reshape(1, -1), W2.astype(jnp.bfloat16),
      b2.reshape(1, -1))


def kernel(w, embeddings, W1, b1, W2, b2):
    B, F = w.shape
    V, E = embeddings.shape
    wlin = w.astype(jnp.int32).reshape(-1)
    x = _sc_gather(embeddings, wlin, B * F, E)
    x2d = x.reshape(B * F * E // 128, 128)
    return _tc_mlp(x2d, W1, b1, W2, b2)


# speedup vs baseline: 3.0999x; 1.0005x over previous
import jax
import jax.numpy as jnp
from jax import lax
from jax.experimental import pallas as pl
from jax.experimental.pallas import tpu as pltpu
from jax.experimental.pallas import tpu_sc as plsc

_BM = 1024


_SUB = 128
_NSUB = 3
_NBUF = 4


def _sc_gather(table, wlin, n_idx, E):
    mesh = plsc.VectorSubcoreMesh(core_axis_name="core", subcore_axis_name="subcore")
    blk = _SUB * _NSUB
    n_steps = n_idx // blk
    info = plsc.get_sparse_core_info()
    nw = info.num_cores * info.num_subcores
    spt = n_steps // nw
    NB = _NBUF

    @pl.kernel(
        out_type=jax.ShapeDtypeStruct((n_steps, blk, E), table.dtype),
        mesh=mesh,
        scratch_types=[
            pltpu.VMEM((NB, blk), jnp.int32),
            pltpu.VMEM((NB, blk, E), jnp.float32),
            pltpu.SemaphoreType.DMA((NB,)),
            pltpu.SemaphoreType.DMA((NB,)),
            pltpu.SemaphoreType.DMA((NB,)),
        ],
        compiler_params=pltpu.CompilerParams(use_tc_tiling_on_sc=False),
    )
    def k(tab_hbm, w_hbm, o_hbm, idx_v, rows_v, isem, gsem, osem):
        wid = lax.axis_index("subcore") * info.num_cores + lax.axis_index("core")
        base = wid * spt

        def idx_cp(s, b):
            return pltpu.make_async_copy(
                w_hbm.at[pl.ds((base + s) * blk, blk)], idx_v.at[b], isem.at[b])

        def gath(b, j):
            return pltpu.make_async_copy(
                tab_hbm.at[idx_v.at[b, pl.ds(j * _SUB, _SUB)]],
                rows_v.at[b, pl.ds(j * _SUB, _SUB), :],
                gsem.at[b])

        def out_cp(s, b):
            return pltpu.make_async_copy(rows_v.at[b], o_hbm.at[base + s], osem.at[b])

        for b in range(NB):
            idx_cp(b, b).start()

        @pl.loop(0, spt // NB)
        def _(oi):
            for b in range(NB):
                s = oi * NB + b
                bp = (b - 1) % NB
                idx_cp(s, b).wait()

                @pl.when(oi > 0)
                def _():
                    out_cp(s - NB, b).wait()

                for j in range(_NSUB):
                    gath(b, j).start()

                @pl.when(s > 0)
                def _():
                    for j in range(_NSUB):
                        gath(bp, j).wait()

                    @pl.when(s - 1 + NB < spt)
                    def _():
                        idx_cp(s - 1 + NB, bp).start()

                    out_cp(s - 1, bp).start()

        bl = (spt - 1) % NB
        for j in range(_NSUB):
            gath(bl, j).wait()
        out_cp(spt - 1, bl).start()
        for b in range(NB):
            out_cp(spt - NB + b, (spt - NB + b) % NB).wait()

    return k(table, wlin)


def _tc_mlp(x2d, W1, b1, W2, b2):
    K, H = W1.shape
    C = W2.shape[1]
    G = K // 128
    B = x2d.shape[0] // G

    def body(x_ref, W1_ref, b1_ref, W2_ref, b2_ref, o_ref):
        acc = jnp.zeros((_BM, H), jnp.float32)
        for g in range(G):
            xg = x_ref[pl.Slice(g, _BM, G), :].astype(jnp.bfloat16)
            acc += jnp.dot(xg, W1_ref[pl.ds(g * 128, 128), :],
                           preferred_element_type=jnp.float32)
        h = jnp.maximum(acc + b1_ref[...], 0.0).astype(jnp.bfloat16)
        o_ref[...] = jnp.dot(h, W2_ref[...], preferred_element_type=jnp.float32) + b2_ref[...]

    return pl.pallas_call(
        body,
        grid=(B // _BM,),
        in_specs=[
            pl.BlockSpec((_BM * G, 128), lambda i: (i, 0)),
            pl.BlockSpec((K, H), lambda i: (0, 0)),
            pl.BlockSpec((1, H), lambda i: (0, 0)),
            pl.BlockSpec((H, C), lambda i: (0, 0)),
            pl.BlockSpec((1, C), lambda i: (0, 0)),
        ],
        out_specs=pl.BlockSpec((_BM, C), lambda i: (i, 0)),
        out_shape=jax.ShapeDtypeStruct((B, C), jnp.float32),
    )(x2d, W1.astype(jnp.bfloat16), b1.reshape(1, -1), W2.astype(jnp.bfloat16),
      b2.reshape(1, -1))


def kernel(w, embeddings, W1, b1, W2, b2):
    B, F = w.shape
    V, E = embeddings.shape
    wlin = w.astype(jnp.int32).reshape(-1)
    x = _sc_gather(embeddings, wlin, B * F, E)
    x2d = x.reshape(B * F * E // 128, 128)
    return _tc_mlp(x2d, W1, b1, W2, b2)


# speedup vs baseline: 3.1074x; 1.0024x over previous
import jax
import jax.numpy as jnp
from jax import lax
from jax.experimental import pallas as pl
from jax.experimental.pallas import tpu as pltpu
from jax.experimental.pallas import tpu_sc as plsc

_BM = 1024


_SUB = 128
_NSUB = 3
_NBUF = 4


def _sc_gather(table, wlin, n_idx, E):
    mesh = plsc.VectorSubcoreMesh(core_axis_name="core", subcore_axis_name="subcore")
    blk = _SUB * _NSUB
    n_steps = n_idx // blk
    info = plsc.get_sparse_core_info()
    nw = info.num_cores * info.num_subcores
    spt = n_steps // nw
    NB = _NBUF

    @pl.kernel(
        out_type=jax.ShapeDtypeStruct((n_steps, blk, E), table.dtype),
        mesh=mesh,
        scratch_types=[
            pltpu.VMEM((NB, blk), jnp.int32),
            pltpu.VMEM((NB, blk, E), jnp.float32),
            pltpu.SemaphoreType.DMA((NB,)),
            pltpu.SemaphoreType.DMA((NB,)),
            pltpu.SemaphoreType.DMA((NB,)),
        ],
        compiler_params=pltpu.CompilerParams(use_tc_tiling_on_sc=False),
    )
    def k(tab_hbm, w_hbm, o_hbm, idx_v, rows_v, isem, gsem, osem):
        wid = lax.axis_index("subcore") * info.num_cores + lax.axis_index("core")
        base = wid * spt

        def idx_cp(s, b):
            return pltpu.make_async_copy(
                w_hbm.at[pl.ds((base + s) * blk, blk)], idx_v.at[b], isem.at[b])

        def gath(b, j):
            return pltpu.make_async_copy(
                tab_hbm.at[idx_v.at[b, pl.ds(j * _SUB, _SUB)]],
                rows_v.at[b, pl.ds(j * _SUB, _SUB), :],
                gsem.at[b])

        def out_cp(s, b):
            return pltpu.make_async_copy(rows_v.at[b], o_hbm.at[base + s], osem.at[b])

        for b in range(NB):
            idx_cp(b, b).start()

        @pl.loop(0, spt // NB)
        def _(oi):
            for b in range(NB):
                s = oi * NB + b
                bp = (b - 1) % NB
                idx_cp(s, b).wait()

                @pl.when(oi > 0)
                def _():
                    out_cp(s - NB, b).wait()

                for j in range(_NSUB):
                    gath(b, j).start()

                @pl.when(s > 0)
                def _():
                    for j in range(_NSUB):
                        gath(bp, j).wait()

                    @pl.when(s - 1 + NB < spt)
                    def _():
                        idx_cp(s - 1 + NB, bp).start()

                    out_cp(s - 1, bp).start()

        bl = (spt - 1) % NB
        for j in range(_NSUB):
            gath(bl, j).wait()
        out_cp(spt - 1, bl).start()
        for b in range(NB):
            out_cp(spt - NB + b, (spt - NB + b) % NB).wait()

    return k(table, wlin)


def _tc_mlp(x2d, W1, b1, W2, b2):
    K, H = W1.shape
    C = W2.shape[1]
    G = K // 128
    B = x2d.shape[0] // G

    def body(x_ref, W1_ref, b1_ref, W2_ref, b2_ref, o_ref):
        acc = jnp.zeros((_BM, H), jnp.float32)
        for g in range(G):
            xg = x_ref[pl.Slice(g, _BM, G), :].astype(jnp.bfloat16)
            acc += jnp.dot(xg, W1_ref[pl.ds(g * 128, 128), :],
                           preferred_element_type=jnp.float32)
        h = jnp.maximum(acc + b1_ref[...], 0.0).astype(jnp.bfloat16)
        o_ref[...] = jnp.dot(h, W2_ref[...], preferred_element_type=jnp.float32) + b2_ref[...]

    return pl.pallas_call(
        body,
        grid=(B // _BM,),
        in_specs=[
            pl.BlockSpec((_BM * G, 128), lambda i: (i, 0)),
            pl.BlockSpec((K, H), lambda i: (0, 0)),
            pl.BlockSpec((1, H), lambda i: (0, 0)),
            pl.BlockSpec((H, C), lambda i: (0, 0)),
            pl.BlockSpec((1, C), lambda i: (0, 0)),
        ],
        out_specs=pl.BlockSpec((_BM, C), lambda i: (i, 0)),
        out_shape=jax.ShapeDtypeStruct((B, C), jnp.float32),
    )(x2d, W1.astype(jnp.bfloat16), b1.reshape(1, -1), W2.astype(jnp.bfloat16),
      b2.reshape(1, -1))


def kernel(w, embeddings, W1, b1, W2, b2):
    B, F = w.shape
    V, E = embeddings.shape
    wlin = w.astype(jnp.int32).reshape(-1)
    half = B * F // 2
    outs = []
    for h in range(2):
        xh = _sc_gather(embeddings, jax.lax.dynamic_slice(wlin, (h * half,), (half,)),
                        half, E)
        x2d = xh.reshape(half * E // 128, 128)
        outs.append(_tc_mlp(x2d, W1, b1, W2, b2))
    return jnp.concatenate(outs, axis=0)
